# baseline (speedup 1.0000x reference)
import functools

import jax
import jax.numpy as jnp
from jax import lax
from jax.experimental import pallas as pl
from jax.experimental.pallas import tpu as pltpu
from jax.experimental.pallas import tpu_sc as plsc

R, C = 4096, 8192
NC, NS, L = 2, 16, 16
NW = NC * NS

S_SC = 768
ROWS_PER_W = S_SC // NW
ROWS_SUB = 4
NCHUNK = ROWS_PER_W // ROWS_SUB
VREGS = C // L
NB = 3

BR = 256
G = 256
R_TC = R - S_SC
OFF_BLK = S_SC // BR

_MESH = plsc.VectorSubcoreMesh(core_axis_name="c", subcore_axis_name="s")


@functools.partial(
    pl.kernel,
    out_type=jax.ShapeDtypeStruct((S_SC, C), jnp.float32),
    mesh=_MESH,
    scratch_types=(
        [pltpu.MemorySpace.VMEM((ROWS_SUB, C), jnp.float32)] * NB
        + [pltpu.SemaphoreType.DMA] * (2 * NB)
    ),
    compiler_params=pltpu.CompilerParams(
        use_tc_tiling_on_sc=False, needs_layout_passes=False
    ),
)
def _cumsum_sc(x_hbm, out_hbm, b0, b1, b2, is0, is1, is2, os0, os1, os2):
    bufs = (b0, b1, b2)
    isems, osems = (is0, is1, is2), (os0, os1, os2)
    wid = lax.axis_index("s") * NC + lax.axis_index("c")
    base = wid * ROWS_PER_W

    def in_desc(k, b):
        r0 = base + k * ROWS_SUB
        return pltpu.make_async_copy(
            x_hbm.at[pl.ds(r0, ROWS_SUB), :], bufs[b], isems[b])

    def out_desc(k, b):
        r0 = base + k * ROWS_SUB
        return pltpu.make_async_copy(
            bufs[b], out_hbm.at[pl.ds(r0, ROWS_SUB), :], osems[b])

    in_desc(0, 0).start()
    if NCHUNK > 1:
        in_desc(1, 1).start()

    for k in range(NCHUNK):
        b = k % NB
        in_desc(k, b).wait()

        def do_vreg(j, carries, buf=bufs[b]):
            c0 = j * L
            new = []
            for r in range(ROWS_SUB):
                v = buf[r, pl.ds(c0, L)]
                s = plsc.cumsum(v)
                t = jnp.sum(v)
                buf[r, pl.ds(c0, L)] = s + carries[r]
                new.append(carries[r] + t)
            return tuple(new)

        lax.fori_loop(0, VREGS, do_vreg,
                      (jnp.float32(0.0),) * ROWS_SUB, unroll=2)
        out_desc(k, b).start()

        if k + 2 < NCHUNK:
            b2 = (k + 2) % NB
            if k >= 1:
                out_desc(k - 1, b2).wait()
            in_desc(k + 2, b2).start()

    for k in range(max(NCHUNK - NB, 0), NCHUNK):
        out_desc(k, k % NB).wait()


def _tc_body(x_ref, o_ref):
    row = lax.broadcasted_iota(jnp.int32, (G, G), 0)
    col = lax.broadcasted_iota(jnp.int32, (G, G), 1)
    tri = jnp.where(row <= col, jnp.float32(1.0), jnp.float32(0.0))

    carry = jnp.zeros((BR, 1), jnp.float32)
    for g in range(C // G):
        blk = x_ref[:, g * G:(g + 1) * G]
        loc = lax.dot_general(blk, tri, (((1,), (0,)), ((), ())),
                              preferred_element_type=jnp.float32)
        out = loc + carry
        o_ref[:, g * G:(g + 1) * G] = out
        carry = out[:, G - 1:G]


def _cumsum_tc(x):
    return pl.pallas_call(
        _tc_body,
        grid=(R_TC // BR,),
        in_specs=[pl.BlockSpec((BR, C), lambda i: (i + OFF_BLK, 0))],
        out_specs=pl.BlockSpec((BR, C), lambda i: (i + OFF_BLK, 0)),
        out_shape=jax.ShapeDtypeStruct((R, C), jnp.float32),
        compiler_params=pltpu.CompilerParams(
            dimension_semantics=("arbitrary",),
        ),
    )(x)


@jax.jit
def kernel(x):
    top = _cumsum_sc(x)
    full = _cumsum_tc(x)
    return lax.dynamic_update_slice(full, top, (0, 0))


# speedup vs baseline: 1.0048x; 1.0048x over previous
import functools

import jax
import jax.numpy as jnp
from jax import lax
from jax.experimental import pallas as pl
from jax.experimental.pallas import tpu as pltpu
from jax.experimental.pallas import tpu_sc as plsc

R, C = 4096, 8192
NC, NS, L = 2, 16, 16
NW = NC * NS

S_SC = 768
ROWS_PER_W = S_SC // NW
ROWS_SUB = 4
NCHUNK = ROWS_PER_W // ROWS_SUB
VREGS = C // L
NB = 3

BR = 256
G = 256
R_TC = R - S_SC
OFF_BLK = S_SC // BR

_MESH = plsc.VectorSubcoreMesh(core_axis_name="c", subcore_axis_name="s")


@functools.partial(
    pl.kernel,
    out_type=jax.ShapeDtypeStruct((S_SC, C), jnp.float32),
    mesh=_MESH,
    scratch_types=(
        [pltpu.MemorySpace.VMEM((ROWS_SUB, C), jnp.float32)] * NB
        + [pltpu.SemaphoreType.DMA] * (2 * NB)
    ),
    compiler_params=pltpu.CompilerParams(
        use_tc_tiling_on_sc=False, needs_layout_passes=False
    ),
)
def _cumsum_sc(x_hbm, out_hbm, b0, b1, b2, is0, is1, is2, os0, os1, os2):
    bufs = (b0, b1, b2)
    isems, osems = (is0, is1, is2), (os0, os1, os2)
    wid = lax.axis_index("s") * NC + lax.axis_index("c")
    base = wid * ROWS_PER_W

    def in_desc(k, b):
        r0 = base + k * ROWS_SUB
        return pltpu.make_async_copy(
            x_hbm.at[pl.ds(r0, ROWS_SUB), :], bufs[b], isems[b])

    def out_desc(k, b):
        r0 = base + k * ROWS_SUB
        return pltpu.make_async_copy(
            bufs[b], out_hbm.at[pl.ds(r0, ROWS_SUB), :], osems[b])

    in_desc(0, 0).start()
    if NCHUNK > 1:
        in_desc(1, 1).start()

    for k in range(NCHUNK):
        b = k % NB
        in_desc(k, b).wait()

        def do_vreg(j, carries, buf=bufs[b]):
            c0 = j * L
            new = []
            for r in range(ROWS_SUB):
                v = buf[r, pl.ds(c0, L)]
                s = plsc.cumsum(v)
                t = jnp.sum(v)
                buf[r, pl.ds(c0, L)] = s + carries[r]
                new.append(carries[r] + t)
            return tuple(new)

        lax.fori_loop(0, VREGS, do_vreg,
                      (jnp.float32(0.0),) * ROWS_SUB, unroll=2)
        out_desc(k, b).start()

        if k + 2 < NCHUNK:
            b2 = (k + 2) % NB
            if k >= 1:
                out_desc(k - 1, b2).wait()
            in_desc(k + 2, b2).start()

    for k in range(max(NCHUNK - NB, 0), NCHUNK):
        out_desc(k, k % NB).wait()


def _tc_body(x_ref, o_ref):
    row = lax.broadcasted_iota(jnp.int32, (G, G), 0)
    col = lax.broadcasted_iota(jnp.int32, (G, G), 1)
    tri = jnp.where(row <= col, jnp.float32(1.0), jnp.float32(0.0))

    carry = jnp.zeros((BR, 1), jnp.float32)
    for g in range(C // G):
        blk = x_ref[:, g * G:(g + 1) * G]
        loc = lax.dot_general(blk, tri, (((1,), (0,)), ((), ())),
                              preferred_element_type=jnp.float32)
        out = loc + carry
        o_ref[:, g * G:(g + 1) * G] = out
        carry = out[:, G - 1:G]


def _cumsum_tc(x):
    return pl.pallas_call(
        _tc_body,
        grid=(R_TC // BR,),
        in_specs=[pl.BlockSpec((BR, C), lambda i: (i + OFF_BLK, 0))],
        out_specs=pl.BlockSpec((BR, C), lambda i: (i + OFF_BLK, 0)),
        out_shape=jax.ShapeDtypeStruct((R, C), jnp.float32),
        compiler_params=pltpu.CompilerParams(
            dimension_semantics=("arbitrary",),
        ),
    )(x)


def _merge_body(full_ref, top_ref, o_ref):
    o_ref[...] = top_ref[...]


def _merge(full, top):
    return pl.pallas_call(
        _merge_body,
        grid=(S_SC // BR,),
        in_specs=[
            pl.BlockSpec(memory_space=pl.MemorySpace.ANY),
            pl.BlockSpec((BR, C), lambda i: (i, 0)),
        ],
        out_specs=pl.BlockSpec((BR, C), lambda i: (i, 0)),
        out_shape=jax.ShapeDtypeStruct((R, C), jnp.float32),
        input_output_aliases={0: 0},
        compiler_params=pltpu.CompilerParams(
            dimension_semantics=("arbitrary",),
        ),
    )(full, top)


@jax.jit
def kernel(x):
    top = _cumsum_sc(x)
    full = _cumsum_tc(x)
    return _merge(full, top)


# speedup vs baseline: 2.0681x; 2.0582x over previous
import functools

import jax
import jax.numpy as jnp
from jax import lax
from jax.experimental import pallas as pl
from jax.experimental.pallas import tpu as pltpu
from jax.experimental.pallas import tpu_sc as plsc

R, C = 4096, 8192
NC, NS, L = 2, 16, 16
NW = NC * NS

S_SC = 768
ROWS_PER_W = S_SC // NW
ROWS_SUB = 8
HALF = C // 2
NCH = (ROWS_PER_W // ROWS_SUB) * 2
VREGS_H = HALF // L
NB = 3

BR = 256
G = 256
R_TC = R - S_SC
OFF_BLK = S_SC // BR

_MESH = plsc.VectorSubcoreMesh(core_axis_name="c", subcore_axis_name="s")


@functools.partial(
    pl.kernel,
    out_type=jax.ShapeDtypeStruct((S_SC, C), jnp.float32),
    mesh=_MESH,
    scratch_types=(
        [pltpu.MemorySpace.VMEM((ROWS_SUB, HALF), jnp.float32)] * NB
        + [pltpu.SemaphoreType.DMA] * (2 * NB)
    ),
    compiler_params=pltpu.CompilerParams(
        use_tc_tiling_on_sc=True, needs_layout_passes=False
    ),
)
def _cumsum_sc(x_hbm, out_hbm, b0, b1, b2, is0, is1, is2, os0, os1, os2):
    bufs = (b0, b1, b2)
    isems, osems = (is0, is1, is2), (os0, os1, os2)
    wid = lax.axis_index("s") * NC + lax.axis_index("c")
    base = wid * ROWS_PER_W

    def slc(q):
        g, h = q // 2, q % 2
        r0 = base + g * ROWS_SUB
        return pl.ds(r0, ROWS_SUB), pl.ds(h * HALF, HALF)

    def in_desc(q, b):
        rs, cs = slc(q)
        return pltpu.make_async_copy(x_hbm.at[rs, cs], bufs[b], isems[b])

    def out_desc(q, b):
        rs, cs = slc(q)
        return pltpu.make_async_copy(bufs[b], out_hbm.at[rs, cs], osems[b])

    in_desc(0, 0).start()
    in_desc(1, 1).start()

    carries = None
    for q in range(NCH):
        b = q % NB
        in_desc(q, b).wait()
        if q % 2 == 0:
            carries = (jnp.float32(0.0),) * ROWS_SUB

        def do_vreg(j, cy, buf=bufs[b]):
            c0 = j * L
            new = []
            for r in range(ROWS_SUB):
                v = buf[r, pl.ds(c0, L)]
                s = plsc.cumsum(v)
                t = jnp.sum(v)
                buf[r, pl.ds(c0, L)] = s + cy[r]
                new.append(cy[r] + t)
            return tuple(new)

        carries = lax.fori_loop(0, VREGS_H, do_vreg, carries)
        out_desc(q, b).start()

        if q + 2 < NCH:
            b2 = (q + 2) % NB
            if q >= 1:
                out_desc(q - 1, b2).wait()
            in_desc(q + 2, b2).start()

    for q in range(NCH - NB, NCH):
        out_desc(q, q % NB).wait()


def _tc_body(x_ref, o_ref):
    row = lax.broadcasted_iota(jnp.int32, (G, G), 0)
    col = lax.broadcasted_iota(jnp.int32, (G, G), 1)
    tri = jnp.where(row <= col, jnp.float32(1.0), jnp.float32(0.0))

    carry = jnp.zeros((BR, 1), jnp.float32)
    for g in range(C // G):
        blk = x_ref[:, g * G:(g + 1) * G]
        loc = lax.dot_general(blk, tri, (((1,), (0,)), ((), ())),
                              preferred_element_type=jnp.float32)
        out = loc + carry
        o_ref[:, g * G:(g + 1) * G] = out
        carry = out[:, G - 1:G]


def _cumsum_tc(x):
    return pl.pallas_call(
        _tc_body,
        grid=(R_TC // BR,),
        in_specs=[pl.BlockSpec((BR, C), lambda i: (i + OFF_BLK, 0))],
        out_specs=pl.BlockSpec((BR, C), lambda i: (i + OFF_BLK, 0)),
        out_shape=jax.ShapeDtypeStruct((R, C), jnp.float32),
        compiler_params=pltpu.CompilerParams(
            dimension_semantics=("arbitrary",),
        ),
    )(x)


def _merge_body(full_ref, top_ref, o_ref):
    o_ref[...] = top_ref[...]


def _merge(full, top):
    return pl.pallas_call(
        _merge_body,
        grid=(S_SC // BR,),
        in_specs=[
            pl.BlockSpec(memory_space=pl.MemorySpace.ANY),
            pl.BlockSpec((BR, C), lambda i: (i, 0)),
        ],
        out_specs=pl.BlockSpec((BR, C), lambda i: (i, 0)),
        out_shape=jax.ShapeDtypeStruct((R, C), jnp.float32),
        input_output_aliases={0: 0},
        compiler_params=pltpu.CompilerParams(
            dimension_semantics=("arbitrary",),
        ),
    )(full, top)


@jax.jit
def kernel(x):
    top = _cumsum_sc(x)
    full = _cumsum_tc(x)
    return _merge(full, top)
